# baseline (speedup 1.0000x reference)
import functools

import jax
import jax.numpy as jnp
import numpy as np
from jax import lax
from jax.experimental import pallas as pl
from jax.experimental.pallas import tpu as pltpu
from jax.experimental.pallas import tpu_sc as plsc

_TEMP = np.float32(0.7)
_INV_TEMP = np.float32(1.0) / _TEMP
_NC, _NS = 2, 16
_NW = _NC * _NS
_L = 16


def _tc_manual(h, w, t_off, T_TC, D, E, RB, NBUF):
    n_blk = T_TC // RB
    assert n_blk >= NBUF and T_TC % RB == 0

    def body(h_hbm, w_ref, o_ref, bufs, sems):
        def start(b, s):
            pltpu.make_async_copy(
                h_hbm.at[pl.ds(t_off + b * RB, RB)], bufs.at[s], sems.at[s]
            ).start()

        def wait(s):
            pltpu.make_async_copy(
                h_hbm.at[pl.ds(t_off, RB)], bufs.at[s], sems.at[s]
            ).wait()

        for s in range(NBUF):
            start(s, s)
        w = w_ref[...]

        def step(b, s):
            wait(s)
            r = lax.dot_general(
                w, bufs[s],
                (((1,), (1,)), ((), ())),
                preferred_element_type=jnp.float32,
            )
            o_ref[:, pl.ds(b * RB, RB)] = r * _INV_TEMP

            @pl.when(b + NBUF < n_blk)
            def _():
                start(b + NBUF, s)

        def grp_body(g, carry):
            for s in range(NBUF):
                step(g * NBUF + s, s)
            return carry

        n_grp = n_blk // NBUF
        lax.fori_loop(0, n_grp, grp_body, 0)
        for i in range(n_grp * NBUF, n_blk):
            step(i, i % NBUF)

    return pl.pallas_call(
        body,
        in_specs=[
            pl.BlockSpec(memory_space=pl.ANY),
            pl.BlockSpec((E, D), lambda: (0, 0)),
        ],
        out_specs=pl.BlockSpec((E, T_TC), lambda: (0, 0)),
        out_shape=jax.ShapeDtypeStruct((E, T_TC), jnp.float32),
        scratch_shapes=[
            pltpu.VMEM((NBUF, RB, D), jnp.float32),
            pltpu.SemaphoreType.DMA((NBUF,)),
        ],
    )(h, w)


def _sc_gate_call(h, w, t_off, T_SC, D, E, CH, TBLK, NBUF=3):
    WT = T_SC // _NW
    n_chunks = WT // CH
    KC = D // _L
    assert WT % CH == 0 and CH % TBLK == 0 and (TBLK * E) % _L == 0

    mesh = plsc.VectorSubcoreMesh(core_axis_name="c", subcore_axis_name="s")

    _dnums = lax.GatherDimensionNumbers(
        offset_dims=(), collapsed_slice_dims=(0,), start_index_map=(0,)
    )

    def _perm(v, idx2d):
        return lax.gather(
            v, idx2d, _dnums, (1,),
            mode=lax.GatherScatterMode.PROMISE_IN_BOUNDS,
        )

    def compute_chunk(buf, w_v, out_v, c, lane, xor_idx):
        for t0 in range(0, CH, TBLK):
            def kbody(k, accs):
                hs = [buf[t0 + i, pl.ds(k * _L, _L)] for i in range(TBLK)]
                new = []
                for e in range(E):
                    wv = w_v[e, pl.ds(k * _L, _L)]
                    for i in range(TBLK):
                        new.append(accs[e * TBLK + i] + hs[i] * wv)
                return tuple(new)

            zero = jnp.zeros((_L,), jnp.float32)
            accs = lax.fori_loop(
                0, KC, kbody, tuple([zero] * (E * TBLK)), unroll=4
            )
            tots = [None] * (TBLK * E)
            for e in range(E):
                for i in range(TBLK):
                    v = accs[e * TBLK + i]
                    for sh in range(4):
                        v = v + _perm(v, xor_idx[sh])
                    tots[i * E + e] = v
            for g in range(TBLK * E // _L):
                res = tots[g * _L]
                for l in range(1, _L):
                    res = jnp.where(lane == l, tots[g * _L + l], res)
                off = (c * CH + t0) * E + g * _L
                out_v[pl.ds(off, _L)] = res * _INV_TEMP

    @functools.partial(
        pl.kernel,
        out_type=jax.ShapeDtypeStruct((T_SC * E,), jnp.float32),
        mesh=mesh,
        scratch_types=[
            pltpu.VMEM((E, D), jnp.float32),
            pltpu.VMEM((NBUF, CH, D), jnp.float32),
            pltpu.VMEM((WT * E,), jnp.float32),
            pltpu.SemaphoreType.DMA((NBUF,)),
        ],
    )
    def k(h_hbm, w_hbm, out_hbm, w_v, buf_v, out_v, sems):
        wid = lax.axis_index("s") * _NC + lax.axis_index("c")
        base = t_off + wid * WT
        lane = lax.iota(jnp.int32, _L)
        xor_idx = [(lane ^ (1 << sh)).reshape(_L, 1) for sh in range(4)]
        pltpu.sync_copy(w_hbm, w_v)

        def start(c, b):
            pltpu.async_copy(
                h_hbm.at[pl.ds(base + c * CH, CH)], buf_v.at[b], sems.at[b]
            )

        def wait(b):
            pltpu.make_async_copy(
                h_hbm.at[pl.ds(base, CH)], buf_v.at[b], sems.at[b]
            ).wait()

        for c in range(min(NBUF, n_chunks)):
            start(c, c)
        for c in range(n_chunks):
            b = c % NBUF
            wait(b)
            compute_chunk(buf_v.at[b], w_v, out_v, c, lane, xor_idx)
            if c + NBUF < n_chunks:
                start(c + NBUF, b)
        pltpu.sync_copy(out_v, out_hbm.at[pl.ds(wid * WT * E, WT * E)])

    return k(h, w)


def kernel(hidden_states, gate_weight):
    B, S, D = hidden_states.shape
    E = gate_weight.shape[0]
    T = B * S
    h = hidden_states.reshape(T, D)

    T_SC = 768
    T_TC = T - T_SC
    out_sc = _sc_gate_call(h, gate_weight, T_TC, T_SC, D, E, CH=8, TBLK=4)
    out_tc = _tc_manual(h, gate_weight, 0, T_TC, D, E, RB=256, NBUF=4)
    out = jnp.concatenate(
        [out_tc, out_sc.reshape(T_SC, E).T], axis=1
    )
    return out.T.reshape(B, S, E)


# speedup vs baseline: 1.4729x; 1.4729x over previous
import functools

import jax
import jax.numpy as jnp
import numpy as np
from jax import lax
from jax.experimental import pallas as pl
from jax.experimental.pallas import tpu as pltpu
from jax.experimental.pallas import tpu_sc as plsc

_TEMP = np.float32(0.7)
_INV_TEMP = np.float32(1.0) / _TEMP
_NC, _NS = 2, 16
_NW = _NC * _NS
_L = 16


def _tc_manual(h, w, t_off, T_TC, D, E, RB, NBUF):
    n_blk = T_TC // RB
    assert n_blk >= NBUF and T_TC % RB == 0

    def body(h_hbm, w_ref, o_ref, bufs, sems):
        def start(b, s):
            pltpu.make_async_copy(
                h_hbm.at[pl.ds(t_off + b * RB, RB)], bufs.at[s], sems.at[s]
            ).start()

        def wait(s):
            pltpu.make_async_copy(
                h_hbm.at[pl.ds(t_off, RB)], bufs.at[s], sems.at[s]
            ).wait()

        for s in range(NBUF):
            start(s, s)
        w = w_ref[...]

        def step(b, s):
            wait(s)
            r = lax.dot_general(
                w, bufs[s],
                (((1,), (1,)), ((), ())),
                preferred_element_type=jnp.float32,
            )
            o_ref[:, pl.ds(b * RB, RB)] = r * _INV_TEMP

            @pl.when(b + NBUF < n_blk)
            def _():
                start(b + NBUF, s)

        def grp_body(g, carry):
            for s in range(NBUF):
                step(g * NBUF + s, s)
            return carry

        n_grp = n_blk // NBUF
        lax.fori_loop(0, n_grp, grp_body, 0)
        for i in range(n_grp * NBUF, n_blk):
            step(i, i % NBUF)

    return pl.pallas_call(
        body,
        in_specs=[
            pl.BlockSpec(memory_space=pl.ANY),
            pl.BlockSpec((E, D), lambda: (0, 0)),
        ],
        out_specs=pl.BlockSpec((E, T_TC), lambda: (0, 0)),
        out_shape=jax.ShapeDtypeStruct((E, T_TC), jnp.float32),
        scratch_shapes=[
            pltpu.VMEM((NBUF, RB, D), jnp.float32),
            pltpu.SemaphoreType.DMA((NBUF,)),
        ],
    )(h, w)


def _sc_gate_call(h, w, t_off, T_SC, D, E, CH, TBLK, NBUF=3):
    WT = T_SC // _NW
    n_chunks = WT // CH
    KC = D // _L
    assert WT % CH == 0 and CH % TBLK == 0 and (TBLK * E) % _L == 0

    mesh = plsc.VectorSubcoreMesh(core_axis_name="c", subcore_axis_name="s")

    _dnums = lax.GatherDimensionNumbers(
        offset_dims=(), collapsed_slice_dims=(0,), start_index_map=(0,)
    )

    def _perm(v, idx2d):
        return lax.gather(
            v, idx2d, _dnums, (1,),
            mode=lax.GatherScatterMode.PROMISE_IN_BOUNDS,
        )

    def compute_chunk(buf, w_v, out_v, c, lane, xor_idx):
        for t0 in range(0, CH, TBLK):
            def kbody(k, accs):
                hs = [buf[t0 + i, pl.ds(k * _L, _L)] for i in range(TBLK)]
                new = []
                for e in range(E):
                    wv = w_v[e, pl.ds(k * _L, _L)]
                    for i in range(TBLK):
                        new.append(accs[e * TBLK + i] + hs[i] * wv)
                return tuple(new)

            zero = jnp.zeros((_L,), jnp.float32)
            accs = lax.fori_loop(
                0, KC, kbody, tuple([zero] * (E * TBLK)), unroll=2
            )
            tots = [None] * (TBLK * E)
            for e in range(E):
                for i in range(TBLK):
                    v = accs[e * TBLK + i]
                    for sh in range(4):
                        v = v + _perm(v, xor_idx[sh])
                    tots[i * E + e] = v
            for g in range(TBLK * E // _L):
                res = tots[g * _L]
                for l in range(1, _L):
                    res = jnp.where(lane == l, tots[g * _L + l], res)
                off = (c * CH + t0) * E + g * _L
                out_v[pl.ds(off, _L)] = res * _INV_TEMP

    @functools.partial(
        pl.kernel,
        out_type=jax.ShapeDtypeStruct((T_SC * E,), jnp.float32),
        mesh=mesh,
        scratch_types=[
            pltpu.VMEM((E, D), jnp.float32),
            pltpu.VMEM((NBUF, CH, D), jnp.float32),
            pltpu.VMEM((WT * E,), jnp.float32),
            pltpu.SemaphoreType.DMA((NBUF,)),
        ],
    )
    def k(h_hbm, w_hbm, out_hbm, w_v, buf_v, out_v, sems):
        wid = lax.axis_index("s") * _NC + lax.axis_index("c")
        base = t_off + wid * WT
        lane = lax.iota(jnp.int32, _L)
        xor_idx = [(lane ^ (1 << sh)).reshape(_L, 1) for sh in range(4)]
        pltpu.sync_copy(w_hbm, w_v)

        def start(c, b):
            pltpu.async_copy(
                h_hbm.at[pl.ds(base + c * CH, CH)], buf_v.at[b], sems.at[b]
            )

        def wait(b):
            pltpu.make_async_copy(
                h_hbm.at[pl.ds(base, CH)], buf_v.at[b], sems.at[b]
            ).wait()

        for c in range(min(NBUF, n_chunks)):
            start(c, c)
        for c in range(n_chunks):
            b = c % NBUF
            wait(b)
            compute_chunk(buf_v.at[b], w_v, out_v, c, lane, xor_idx)
            if c + NBUF < n_chunks:
                start(c + NBUF, b)
        pltpu.sync_copy(out_v, out_hbm.at[pl.ds(wid * WT * E, WT * E)])

    return k(h, w)


def kernel(hidden_states, gate_weight):
    B, S, D = hidden_states.shape
    E = gate_weight.shape[0]
    T = B * S
    h = hidden_states.reshape(T, D)

    T_SC = 256
    T_TC = T - T_SC
    out_sc = _sc_gate_call(h, gate_weight, T_TC, T_SC, D, E, CH=8, TBLK=4)
    out_tc = _tc_manual(h, gate_weight, 0, T_TC, D, E, RB=256, NBUF=4)
    out = jnp.concatenate(
        [out_tc, out_sc.reshape(T_SC, E).T], axis=1
    )
    return out.T.reshape(B, S, E)


# speedup vs baseline: 2.6748x; 1.8159x over previous
import functools

import jax
import jax.numpy as jnp
import numpy as np
from jax import lax
from jax.experimental import pallas as pl
from jax.experimental.pallas import tpu as pltpu
from jax.experimental.pallas import tpu_sc as plsc

_TEMP = np.float32(0.7)
_INV_TEMP = np.float32(1.0) / _TEMP
_NC, _NS = 2, 16
_NW = _NC * _NS
_L = 16


def _tc_manual(h, w, t_off, T_TC, D, E, RB, NBUF):
    n_blk = T_TC // RB
    assert n_blk >= NBUF and T_TC % RB == 0

    def body(h_hbm, w_ref, o_ref, bufs, sems):
        def start(b, s):
            pltpu.make_async_copy(
                h_hbm.at[pl.ds(t_off + b * RB, RB)], bufs.at[s], sems.at[s]
            ).start()

        def wait(s):
            pltpu.make_async_copy(
                h_hbm.at[pl.ds(t_off, RB)], bufs.at[s], sems.at[s]
            ).wait()

        for s in range(NBUF):
            start(s, s)
        w = w_ref[...]

        def step(b, s):
            wait(s)
            r = lax.dot_general(
                w, bufs[s],
                (((1,), (1,)), ((), ())),
                preferred_element_type=jnp.float32,
            )
            o_ref[:, pl.ds(b * RB, RB)] = r * _INV_TEMP

            @pl.when(b + NBUF < n_blk)
            def _():
                start(b + NBUF, s)

        def grp_body(g, carry):
            for s in range(NBUF):
                step(g * NBUF + s, s)
            return carry

        n_grp = n_blk // NBUF
        lax.fori_loop(0, n_grp, grp_body, 0)
        for i in range(n_grp * NBUF, n_blk):
            step(i, i % NBUF)

    return pl.pallas_call(
        body,
        in_specs=[
            pl.BlockSpec(memory_space=pl.ANY),
            pl.BlockSpec((E, D), lambda: (0, 0)),
        ],
        out_specs=pl.BlockSpec((E, T_TC), lambda: (0, 0)),
        out_shape=jax.ShapeDtypeStruct((E, T_TC), jnp.float32),
        scratch_shapes=[
            pltpu.VMEM((NBUF, RB, D), jnp.float32),
            pltpu.SemaphoreType.DMA((NBUF,)),
        ],
    )(h, w)


def _sc_gate_call(h, w, t_off, T_SC, D, E, CH, TBLK, NBUF=3):
    WT = T_SC // _NW
    n_chunks = WT // CH
    KC = D // _L
    assert WT % CH == 0 and CH % TBLK == 0 and (TBLK * E) % _L == 0

    mesh = plsc.VectorSubcoreMesh(core_axis_name="c", subcore_axis_name="s")

    _dnums = lax.GatherDimensionNumbers(
        offset_dims=(), collapsed_slice_dims=(0,), start_index_map=(0,)
    )

    def _perm(v, idx2d):
        return lax.gather(
            v, idx2d, _dnums, (1,),
            mode=lax.GatherScatterMode.PROMISE_IN_BOUNDS,
        )

    def compute_chunk(buf, w_v, out_v, c, lane, xor_idx):
        for t0 in range(0, CH, TBLK):
            def kbody(k, accs):
                hs = [buf[t0 + i, pl.ds(k * _L, _L)] for i in range(TBLK)]
                new = []
                for e in range(E):
                    wv = w_v[e, pl.ds(k * _L, _L)]
                    for i in range(TBLK):
                        new.append(accs[e * TBLK + i] + hs[i] * wv)
                return tuple(new)

            zero = jnp.zeros((_L,), jnp.float32)
            accs = lax.fori_loop(
                0, KC, kbody, tuple([zero] * (E * TBLK)), unroll=2
            )
            tots = [None] * (TBLK * E)
            for e in range(E):
                for i in range(TBLK):
                    v = accs[e * TBLK + i]
                    for sh in range(4):
                        v = v + _perm(v, xor_idx[sh])
                    tots[i * E + e] = v
            for g in range(TBLK * E // _L):
                res = tots[g * _L]
                for l in range(1, _L):
                    res = jnp.where(lane == l, tots[g * _L + l], res)
                off = (c * CH + t0) * E + g * _L
                out_v[pl.ds(off, _L)] = res * _INV_TEMP

    @functools.partial(
        pl.kernel,
        out_type=jax.ShapeDtypeStruct((T_SC * E,), jnp.float32),
        mesh=mesh,
        scratch_types=[
            pltpu.VMEM((E, D), jnp.float32),
            pltpu.VMEM((NBUF, CH, D), jnp.float32),
            pltpu.VMEM((WT * E,), jnp.float32),
            pltpu.SemaphoreType.DMA((NBUF,)),
        ],
    )
    def k(h_hbm, w_hbm, out_hbm, w_v, buf_v, out_v, sems):
        wid = lax.axis_index("s") * _NC + lax.axis_index("c")
        base = t_off + wid * WT
        lane = lax.iota(jnp.int32, _L)
        xor_idx = [(lane ^ (1 << sh)).reshape(_L, 1) for sh in range(4)]
        pltpu.sync_copy(w_hbm, w_v)

        def start(c, b):
            pltpu.async_copy(
                h_hbm.at[pl.ds(base + c * CH, CH)], buf_v.at[b], sems.at[b]
            )

        def wait(b):
            pltpu.make_async_copy(
                h_hbm.at[pl.ds(base, CH)], buf_v.at[b], sems.at[b]
            ).wait()

        for c in range(min(NBUF, n_chunks)):
            start(c, c)
        for c in range(n_chunks):
            b = c % NBUF
            wait(b)
            compute_chunk(buf_v.at[b], w_v, out_v, c, lane, xor_idx)
            if c + NBUF < n_chunks:
                start(c + NBUF, b)
        pltpu.sync_copy(out_v, out_hbm.at[pl.ds(wid * WT * E, WT * E)])

    return k(h, w)


def kernel(hidden_states, gate_weight):
    B, S, D = hidden_states.shape
    E = gate_weight.shape[0]
    T = B * S
    h = hidden_states.reshape(T, D)

    out = _tc_manual(h, gate_weight, 0, T, D, E, RB=512, NBUF=4)
    return out.T.reshape(B, S, E)


# speedup vs baseline: 2.6859x; 1.0041x over previous
import functools

import jax
import jax.numpy as jnp
import numpy as np
from jax import lax
from jax.experimental import pallas as pl
from jax.experimental.pallas import tpu as pltpu
from jax.experimental.pallas import tpu_sc as plsc

_TEMP = np.float32(0.7)
_INV_TEMP = np.float32(1.0) / _TEMP
_NC, _NS = 2, 16
_NW = _NC * _NS
_L = 16


def _tc_manual(h, w, t_off, T_TC, D, E, RB, NBUF):
    n_blk = T_TC // RB
    assert n_blk >= NBUF and T_TC % RB == 0

    def body(h_hbm, w_ref, o_ref, bufs, sems):
        def start(b, s):
            pltpu.make_async_copy(
                h_hbm.at[pl.ds(t_off + b * RB, RB)], bufs.at[s], sems.at[s]
            ).start()

        def wait(s):
            pltpu.make_async_copy(
                h_hbm.at[pl.ds(t_off, RB)], bufs.at[s], sems.at[s]
            ).wait()

        for s in range(NBUF):
            start(s, s)
        w = w_ref[...]

        def step(b, s):
            wait(s)
            r = lax.dot_general(
                w, bufs[s],
                (((1,), (1,)), ((), ())),
                preferred_element_type=jnp.float32,
            )
            o_ref[:, pl.ds(b * RB, RB)] = r * _INV_TEMP

            @pl.when(b + NBUF < n_blk)
            def _():
                start(b + NBUF, s)

        def grp_body(g, carry):
            for s in range(NBUF):
                step(g * NBUF + s, s)
            return carry

        n_grp = n_blk // NBUF
        lax.fori_loop(0, n_grp, grp_body, 0)
        for i in range(n_grp * NBUF, n_blk):
            step(i, i % NBUF)

    return pl.pallas_call(
        body,
        in_specs=[
            pl.BlockSpec(memory_space=pl.ANY),
            pl.BlockSpec((E, D), lambda: (0, 0)),
        ],
        out_specs=pl.BlockSpec((E, T_TC), lambda: (0, 0)),
        out_shape=jax.ShapeDtypeStruct((E, T_TC), jnp.float32),
        scratch_shapes=[
            pltpu.VMEM((NBUF, RB, D), jnp.float32),
            pltpu.SemaphoreType.DMA((NBUF,)),
        ],
    )(h, w)


def _sc_gate_call(h, w, t_off, T_SC, D, E, CH, TBLK, NBUF=3):
    WT = T_SC // _NW
    n_chunks = WT // CH
    KC = D // _L
    assert WT % CH == 0 and CH % TBLK == 0 and (TBLK * E) % _L == 0

    mesh = plsc.VectorSubcoreMesh(core_axis_name="c", subcore_axis_name="s")

    _dnums = lax.GatherDimensionNumbers(
        offset_dims=(), collapsed_slice_dims=(0,), start_index_map=(0,)
    )

    def _perm(v, idx2d):
        return lax.gather(
            v, idx2d, _dnums, (1,),
            mode=lax.GatherScatterMode.PROMISE_IN_BOUNDS,
        )

    def compute_chunk(buf, w_v, out_v, c, lane, xor_idx):
        for t0 in range(0, CH, TBLK):
            def kbody(k, accs):
                hs = [buf[t0 + i, pl.ds(k * _L, _L)] for i in range(TBLK)]
                new = []
                for e in range(E):
                    wv = w_v[e, pl.ds(k * _L, _L)]
                    for i in range(TBLK):
                        new.append(accs[e * TBLK + i] + hs[i] * wv)
                return tuple(new)

            zero = jnp.zeros((_L,), jnp.float32)
            accs = lax.fori_loop(
                0, KC, kbody, tuple([zero] * (E * TBLK)), unroll=2
            )
            tots = [None] * (TBLK * E)
            for e in range(E):
                for i in range(TBLK):
                    v = accs[e * TBLK + i]
                    for sh in range(4):
                        v = v + _perm(v, xor_idx[sh])
                    tots[i * E + e] = v
            for g in range(TBLK * E // _L):
                res = tots[g * _L]
                for l in range(1, _L):
                    res = jnp.where(lane == l, tots[g * _L + l], res)
                off = (c * CH + t0) * E + g * _L
                out_v[pl.ds(off, _L)] = res * _INV_TEMP

    @functools.partial(
        pl.kernel,
        out_type=jax.ShapeDtypeStruct((T_SC * E,), jnp.float32),
        mesh=mesh,
        scratch_types=[
            pltpu.VMEM((E, D), jnp.float32),
            pltpu.VMEM((NBUF, CH, D), jnp.float32),
            pltpu.VMEM((WT * E,), jnp.float32),
            pltpu.SemaphoreType.DMA((NBUF,)),
        ],
    )
    def k(h_hbm, w_hbm, out_hbm, w_v, buf_v, out_v, sems):
        wid = lax.axis_index("s") * _NC + lax.axis_index("c")
        base = t_off + wid * WT
        lane = lax.iota(jnp.int32, _L)
        xor_idx = [(lane ^ (1 << sh)).reshape(_L, 1) for sh in range(4)]
        pltpu.sync_copy(w_hbm, w_v)

        def start(c, b):
            pltpu.async_copy(
                h_hbm.at[pl.ds(base + c * CH, CH)], buf_v.at[b], sems.at[b]
            )

        def wait(b):
            pltpu.make_async_copy(
                h_hbm.at[pl.ds(base, CH)], buf_v.at[b], sems.at[b]
            ).wait()

        for c in range(min(NBUF, n_chunks)):
            start(c, c)
        for c in range(n_chunks):
            b = c % NBUF
            wait(b)
            compute_chunk(buf_v.at[b], w_v, out_v, c, lane, xor_idx)
            if c + NBUF < n_chunks:
                start(c + NBUF, b)
        pltpu.sync_copy(out_v, out_hbm.at[pl.ds(wid * WT * E, WT * E)])

    return k(h, w)


def kernel(hidden_states, gate_weight):
    B, S, D = hidden_states.shape
    E = gate_weight.shape[0]
    T = B * S
    h = hidden_states.reshape(T, D)

    out = _tc_manual(h, gate_weight, 0, T, D, E, RB=1024, NBUF=3)
    return out.T.reshape(B, S, E)
